# Initial kernel scaffold; baseline (speedup 1.0000x reference)
import jax
import jax.numpy as jnp
from jax.experimental import pallas as pl


def kernel(x, edge_index, W1, b1, W2, b2):
    raise NotImplementedError("write your pallas kernel here")


# speedup vs baseline: 24.1622x; 24.1622x over previous
import functools

import jax
import jax.numpy as jnp
from jax import lax
from jax.experimental import pallas as pl
from jax.experimental.pallas import tpu as pltpu
from jax.experimental.pallas import tpu_sc as plsc

N = 10000
E = 320000
F_IN = 128
HID = 128
C = 64

NP = 10240
NCORE = 2
NSUB = 16
NWORK = NCORE * NSUB
EPW = E // NWORK
CHUNK = 80
NCH = EPW // CHUNK
RPS = NP // NSUB

_MESH = plsc.VectorSubcoreMesh(core_axis_name="c", subcore_axis_name="s")


def _deg_body(dst_hbm, out_hbm, dst_v, deg_v):
    c = lax.axis_index("c")
    s = lax.axis_index("s")
    wid = c * NSUB + s
    pltpu.sync_copy(dst_hbm.at[pl.ds(wid * EPW, EPW)], dst_v)

    zeros16 = jnp.zeros((16,), jnp.float32)

    def _zero(i, carry):
        deg_v[pl.ds(i * 16, 16)] = zeros16
        return carry

    lax.fori_loop(0, NP // 16, _zero, 0)

    ones16 = jnp.ones((16,), jnp.float32)

    def _count(i, carry):
        idx = dst_v[pl.ds(i * 16, 16)]
        plsc.addupdate_scatter(deg_v, [idx], ones16)
        return carry

    lax.fori_loop(0, EPW // 16, _count, 0)
    pltpu.sync_copy(deg_v, out_hbm.at[c, s])


_deg_call = functools.partial(
    pl.kernel,
    out_type=jax.ShapeDtypeStruct((NCORE, NSUB, NP), jnp.float32),
    mesh=_MESH,
    scratch_types=[
        pltpu.VMEM((EPW,), jnp.int32),
        pltpu.VMEM((NP,), jnp.float32),
    ],
    compiler_params=pltpu.CompilerParams(needs_layout_passes=False),
)(_deg_body)


def _agg_body(table_hbm, src_hbm, dst_hbm, out_hbm,
              src_i, dst_i, buf, acc, sem0, sem1, D):
    c = lax.axis_index("c")
    s = lax.axis_index("s")
    wid = c * NSUB + s
    pltpu.sync_copy(src_hbm.at[pl.ds(wid * NCH, NCH)], src_i)
    pltpu.sync_copy(dst_hbm.at[pl.ds(wid * NCH, NCH)], dst_i)

    zeros16 = jnp.zeros((16,), jnp.float32)
    lanes = D // 16

    def _zero(i, carry):
        buf[0, i // lanes, pl.ds((i % lanes) * 16, 16)] = zeros16
        return carry

    lax.fori_loop(0, CHUNK * lanes, _zero, 0)

    def _zstripe(k, carry):
        pltpu.sync_copy(buf.at[0], acc.at[pl.ds(s * RPS + k * CHUNK, CHUNK)])
        return carry

    lax.fori_loop(0, RPS // CHUNK, _zstripe, 0)
    plsc.subcore_barrier()

    def _g_start(j, slot, sem):
        pltpu.make_async_copy(table_hbm.at[src_i.at[j]], buf.at[slot], sem).start()

    def _g_wait(slot, sem):
        pltpu.make_async_copy(table_hbm.at[src_i.at[0]], buf.at[slot], sem).wait()

    def _scat(j, slot):
        pltpu.sync_copy(buf.at[slot], acc.at[dst_i.at[j]], add=True)

    _g_start(0, 0, sem0)

    def _body(jj, carry):
        j = jj * 2
        _g_wait(0, sem0)
        _g_start(j + 1, 1, sem1)
        _scat(j, 0)
        _g_wait(1, sem1)
        _g_start(j + 2, 0, sem0)
        _scat(j + 1, 1)
        return carry

    lax.fori_loop(0, (NCH - 1) // 2, _body, 0)
    _g_wait(0, sem0)
    _scat(NCH - 1, 0)

    plsc.subcore_barrier()

    def _out(k, carry):
        rows = pl.ds(s * RPS + k * CHUNK, CHUNK)
        pltpu.sync_copy(acc.at[rows], buf.at[0])
        pltpu.sync_copy(buf.at[0], out_hbm.at[c, rows])
        return carry

    lax.fori_loop(0, RPS // CHUNK, _out, 0)


def _make_agg(D):
    return functools.partial(
        pl.kernel,
        out_type=jax.ShapeDtypeStruct((NCORE, NP, D), jnp.float32),
        mesh=_MESH,
        scratch_types=[
            pltpu.VMEM((NCH, CHUNK), jnp.int32),
            pltpu.VMEM((NCH, CHUNK), jnp.int32),
            pltpu.VMEM((2, CHUNK, D), jnp.float32),
            pltpu.VMEM_SHARED((NP, D), jnp.float32),
            pltpu.SemaphoreType.DMA,
            pltpu.SemaphoreType.DMA,
        ],
        compiler_params=pltpu.CompilerParams(
            needs_layout_passes=False, use_tc_tiling_on_sc=False),
    )(functools.partial(_agg_body, D=D))


_agg_hid = _make_agg(HID)
_agg_c = _make_agg(C)


_BR = 256


def _prep_body(degp_ref, x_ref, w1_ref, dis_ref, h1_ref):
    cnt = jnp.sum(degp_ref[...], axis=(0, 1))
    dis = lax.rsqrt(cnt + 1.0)
    dis_ref[...] = dis
    h1_ref[...] = jnp.dot(x_ref[...], w1_ref[...],
                          preferred_element_type=jnp.float32) * dis[:, None]


def _prep_call(degp, xp, W1):
    grid = NP // _BR
    return pl.pallas_call(
        _prep_body,
        grid=(grid,),
        in_specs=[
            pl.BlockSpec((NCORE, NSUB, _BR), lambda i: (0, 0, i)),
            pl.BlockSpec((_BR, F_IN), lambda i: (i, 0)),
            pl.BlockSpec((F_IN, HID), lambda i: (0, 0)),
        ],
        out_specs=[
            pl.BlockSpec((_BR,), lambda i: (i,)),
            pl.BlockSpec((_BR, HID), lambda i: (i, 0)),
        ],
        out_shape=[
            jax.ShapeDtypeStruct((NP,), jnp.float32),
            jax.ShapeDtypeStruct((NP, HID), jnp.float32),
        ],
    )(degp, xp, W1)


def _mid_body(aggp_ref, h1_ref, dis_ref, b1_ref, w2_ref, h2_ref):
    dis = dis_ref[...][:, None]
    t = (aggp_ref[0] + aggp_ref[1] + h1_ref[...]) * dis + b1_ref[...]
    t = jnp.maximum(t, 0.0)
    h2_ref[...] = jnp.dot(t, w2_ref[...],
                          preferred_element_type=jnp.float32) * dis


def _mid_call(aggp, h1p, dis, b1r, W2):
    grid = NP // _BR
    return pl.pallas_call(
        _mid_body,
        grid=(grid,),
        in_specs=[
            pl.BlockSpec((NCORE, _BR, HID), lambda i: (0, i, 0)),
            pl.BlockSpec((_BR, HID), lambda i: (i, 0)),
            pl.BlockSpec((_BR,), lambda i: (i,)),
            pl.BlockSpec((1, HID), lambda i: (0, 0)),
            pl.BlockSpec((HID, C), lambda i: (0, 0)),
        ],
        out_specs=pl.BlockSpec((_BR, C), lambda i: (i, 0)),
        out_shape=jax.ShapeDtypeStruct((NP, C), jnp.float32),
    )(aggp, h1p, dis, b1r, W2)


def _fin_body(aggp_ref, h2_ref, dis_ref, b2_ref, z_ref):
    dis = dis_ref[...][:, None]
    z_ref[...] = (aggp_ref[0] + aggp_ref[1] + h2_ref[...]) * dis + b2_ref[...]


def _fin_call(aggp, h2p, dis, b2r):
    grid = NP // _BR
    return pl.pallas_call(
        _fin_body,
        grid=(grid,),
        in_specs=[
            pl.BlockSpec((NCORE, _BR, C), lambda i: (0, i, 0)),
            pl.BlockSpec((_BR, C), lambda i: (i, 0)),
            pl.BlockSpec((_BR,), lambda i: (i,)),
            pl.BlockSpec((1, C), lambda i: (0, 0)),
        ],
        out_specs=pl.BlockSpec((_BR, C), lambda i: (i, 0)),
        out_shape=jax.ShapeDtypeStruct((NP, C), jnp.float32),
    )(aggp, h2p, dis, b2r)


def kernel(x, edge_index, W1, b1, W2, b2):
    src2 = edge_index[0].reshape(NWORK * NCH, CHUNK)
    dst2 = edge_index[1].reshape(NWORK * NCH, CHUNK)
    xp = jnp.pad(x, ((0, NP - N), (0, 0)))

    degp = _deg_call(edge_index[1])
    dis, h1p = _prep_call(degp, xp, W1)
    agg1 = _agg_hid(h1p, src2, dst2)
    h2p = _mid_call(agg1, h1p, dis, b1.reshape(1, HID), W2)
    agg2 = _agg_c(h2p, src2, dst2)
    zp = _fin_call(agg2, h2p, dis, b2.reshape(1, C))
    return zp[:N]
